# baseline (speedup 1.0000x reference)
import functools

import jax
import jax.numpy as jnp
from jax import lax
from jax.experimental import pallas as pl
from jax.experimental.pallas import tpu as pltpu
from jax.experimental.pallas import tpu_sc as plsc

N = 10000
E = 320000
D = 128
H = 640
B = 128
OUT_DIM = 64

NC = 2
NS = 16
NW = NC * NS
EPW = E // NW
CHUNK = 125
NCHUNK = EPW // CHUNK
NPAD = 10240
RPT = NPAD // NS
RB = 1000
NRB = N // RB
SLOPE = 0.01


def _lrelu(x):
    return jnp.where(x >= 0, x, SLOPE * x)


def _deg_body(dst1_h, dst2_h, deg_h, idx_v, ones_v, zero_v, deg_sh, dsem):
    cid = lax.axis_index("c")
    sid = lax.axis_index("s")
    wid = cid * NS + sid
    onerow = jnp.where(lax.iota(jnp.int32, 16) == 0,
                       jnp.full((16,), 1.0, jnp.float32),
                       jnp.zeros((16,), jnp.float32))
    zrow = jnp.zeros((16,), jnp.float32)

    def init_ones(i, carry):
        ones_v[i, :] = onerow
        return carry

    def init_zero(i, carry):
        zero_v[i, :] = zrow
        return carry

    lax.fori_loop(0, CHUNK, init_ones, 0)
    lax.fori_loop(0, 64, init_zero, 0)

    def zero_acc():
        for k in range(RPT // 64):
            pltpu.sync_copy(zero_v, deg_sh.at[pl.ds(sid * RPT + k * 64, 64)])

    zero_acc()
    plsc.subcore_barrier()

    for ch in range(2):
        dst_h = dst1_h if ch == 0 else dst2_h
        pltpu.sync_copy(dst_h.at[pl.ds(wid * NCHUNK, NCHUNK)], idx_v)

        def chunk_fire(j, carry):
            pltpu.async_copy(ones_v, deg_sh.at[idx_v.at[j]], dsem, add=True)
            return carry

        def chunk_drain(j, carry):
            pltpu.make_async_copy(ones_v, deg_sh.at[idx_v.at[j]],
                                  dsem).wait()
            return carry

        lax.fori_loop(0, NCHUNK, chunk_fire, 0)
        lax.fori_loop(0, NCHUNK, chunk_drain, 0)
        plsc.subcore_barrier()
        pltpu.sync_copy(deg_sh.at[pl.ds(sid * RPT, RPT)],
                        deg_h.at[ch, cid, pl.ds(sid * RPT, RPT)])
        plsc.subcore_barrier()
        if ch == 0:
            zero_acc()
            plsc.subcore_barrier()


def _scatter_body(src1_h, dst1_h, src2_h, dst2_h, y1_h, y2_h, part_h,
                  sidx_v, didx_v, rows0_v, rows1_v, zero_v, acc_sh,
                  sem0, sem1, ssem0, ssem1):
    cid = lax.axis_index("c")
    sid = lax.axis_index("s")
    wid = cid * NS + sid
    zrow = jnp.zeros((16,), jnp.float32)

    def zero_rows(i, carry):
        for jc in range(D // 16):
            zero_v[i, pl.ds(jc * 16, 16)] = zrow
        return carry

    lax.fori_loop(0, 16, zero_rows, 0)

    def zero_acc():
        for k in range(RPT // 16):
            pltpu.sync_copy(zero_v, acc_sh.at[pl.ds(sid * RPT + k * 16, 16)])

    zero_acc()
    plsc.subcore_barrier()

    for ch in range(2):
        src_h = src1_h if ch == 0 else src2_h
        dst_h = dst1_h if ch == 0 else dst2_h
        y_h = y1_h if ch == 0 else y2_h
        for half in range(2):
            hbase = wid * NCHUNK + half * (NCHUNK // 2)
            pltpu.sync_copy(src_h.at[pl.ds(hbase, NCHUNK // 2)], sidx_v)
            pltpu.sync_copy(dst_h.at[pl.ds(hbase, NCHUNK // 2)], didx_v)

            npair = NCHUNK // 4
            pltpu.async_copy(y_h.at[sidx_v.at[0]], rows0_v, sem0)
            pltpu.async_copy(y_h.at[sidx_v.at[1]], rows1_v, sem1)

            def pair_body(k, carry):
                j0 = 2 * k
                pltpu.make_async_copy(
                    y_h.at[sidx_v.at[j0]], rows0_v, sem0).wait()
                pltpu.async_copy(
                    rows0_v, acc_sh.at[didx_v.at[j0]], ssem0, add=True)
                pltpu.make_async_copy(
                    y_h.at[sidx_v.at[j0 + 1]], rows1_v, sem1).wait()
                pltpu.async_copy(
                    rows1_v, acc_sh.at[didx_v.at[j0 + 1]], ssem1, add=True)
                pltpu.make_async_copy(
                    rows0_v, acc_sh.at[didx_v.at[j0]], ssem0).wait()
                pltpu.async_copy(y_h.at[sidx_v.at[j0 + 2]], rows0_v, sem0)
                pltpu.make_async_copy(
                    rows1_v, acc_sh.at[didx_v.at[j0 + 1]], ssem1).wait()
                pltpu.async_copy(y_h.at[sidx_v.at[j0 + 3]], rows1_v, sem1)
                return carry

            lax.fori_loop(0, npair - 1, pair_body, 0)
            jlast = NCHUNK // 2 - 2
            pltpu.make_async_copy(
                y_h.at[sidx_v.at[jlast]], rows0_v, sem0).wait()
            pltpu.async_copy(
                rows0_v, acc_sh.at[didx_v.at[jlast]], ssem0, add=True)
            pltpu.make_async_copy(
                y_h.at[sidx_v.at[jlast + 1]], rows1_v, sem1).wait()
            pltpu.async_copy(
                rows1_v, acc_sh.at[didx_v.at[jlast + 1]], ssem1, add=True)
            pltpu.make_async_copy(
                rows0_v, acc_sh.at[didx_v.at[jlast]], ssem0).wait()
            pltpu.make_async_copy(
                rows1_v, acc_sh.at[didx_v.at[jlast + 1]], ssem1).wait()
        plsc.subcore_barrier()
        pltpu.sync_copy(acc_sh.at[pl.ds(sid * RPT, RPT)],
                        part_h.at[ch, cid, pl.ds(sid * RPT, RPT)])
        plsc.subcore_barrier()
        if ch == 0:
            zero_acc()
            plsc.subcore_barrier()


def _y_body(x1_ref, x2_ref, dp_ref, y1_ref, y2_ref):
    dp = dp_ref[...]
    for ch in range(2):
        deg = dp[ch, 0, :, 0:1] + dp[ch, 1, :, 0:1] + 1.0
        dinv = lax.rsqrt(deg)
        x = (x1_ref if ch == 0 else x2_ref)[...]
        (y1_ref if ch == 0 else y2_ref)[...] = x * dinv


def _main_body(part_ref, y1_ref, y2_ref, dp_ref, bf_ref, W_ref, b_ref,
               fcW_ref, fcb_ref, f1W_ref, f1b_ref, f2W_ref, f2b_ref,
               oW_ref, ob_ref, out_ref, pooled_sc, cnt_sc):
    r = pl.program_id(0)

    @pl.when(r == 0)
    def _init():
        pooled_sc[...] = jnp.zeros_like(pooled_sc)
        cnt_sc[...] = jnp.zeros_like(cnt_sc)

    iota_g = lax.broadcasted_iota(jnp.int32, (RB, B), 1).astype(jnp.float32)
    ones_col = jnp.ones((RB, 8), jnp.float32)
    dp = dp_ref[...]
    for ch in range(2):
        yv = (y1_ref if ch == 0 else y2_ref)[...]
        deg = dp[ch, 0, :, 0:1] + dp[ch, 1, :, 0:1] + 1.0
        dinv = lax.rsqrt(deg)
        agg = dinv * (part_ref[ch, 0] + part_ref[ch, 1] + yv)
        h = jnp.dot(agg, W_ref[ch], preferred_element_type=jnp.float32)
        z = _lrelu(h + b_ref[ch])
        oh = jnp.where(bf_ref[ch] == iota_g, 1.0, 0.0)
        pooled_sc[ch] += lax.dot_general(
            oh, z, (((0,), (0,)), ((), ())), preferred_element_type=jnp.float32)
        cnt_sc[ch] += lax.dot_general(
            oh, ones_col, (((0,), (0,)), ((), ())),
            preferred_element_type=jnp.float32)

    @pl.when(r == NRB - 1)
    def _final():
        feats = []
        for ch in range(2):
            cnt = jnp.maximum(cnt_sc[ch, :, 0:1], 1.0)
            mean = pooled_sc[ch] / cnt
            t = jnp.dot(mean, fcW_ref[ch],
                        preferred_element_type=jnp.float32) + fcb_ref[ch]
            feats.append(_lrelu(t))
        xc = jnp.concatenate(feats, axis=1)
        u = _lrelu(jnp.dot(xc, f1W_ref[...],
                           preferred_element_type=jnp.float32) + f1b_ref[...])
        v = _lrelu(jnp.dot(u, f2W_ref[...],
                           preferred_element_type=jnp.float32) + f2b_ref[...])
        out_ref[...] = jnp.dot(v, oW_ref[...],
                               preferred_element_type=jnp.float32) + ob_ref[...]


def _sc_mesh():
    return plsc.VectorSubcoreMesh(core_axis_name="c", subcore_axis_name="s",
                                  num_cores=NC, num_subcores=NS)


def _deg_call(dst1, dst2):
    f = pl.kernel(
        _deg_body,
        out_type=jax.ShapeDtypeStruct((2, NC, NPAD, 16), jnp.float32),
        mesh=_sc_mesh(),
        scratch_types=[
            pltpu.VMEM((NCHUNK, CHUNK), jnp.int32),
            pltpu.VMEM((CHUNK, 16), jnp.float32),
            pltpu.VMEM((64, 16), jnp.float32),
            pltpu.VMEM_SHARED((NPAD, 16), jnp.float32),
            pltpu.SemaphoreType.DMA,
        ],
    )
    return f(dst1, dst2)


def _scatter_call(src1, dst1, src2, dst2, y1, y2):
    f = pl.kernel(
        _scatter_body,
        out_type=jax.ShapeDtypeStruct((2, NC, NPAD, D), jnp.float32),
        mesh=_sc_mesh(),
        scratch_types=[
            pltpu.VMEM((NCHUNK // 2, CHUNK), jnp.int32),
            pltpu.VMEM((NCHUNK // 2, CHUNK), jnp.int32),
            pltpu.VMEM((CHUNK, D), jnp.float32),
            pltpu.VMEM((CHUNK, D), jnp.float32),
            pltpu.VMEM((16, D), jnp.float32),
            pltpu.VMEM_SHARED((NPAD, D), jnp.float32),
            pltpu.SemaphoreType.DMA,
            pltpu.SemaphoreType.DMA,
            pltpu.SemaphoreType.DMA,
            pltpu.SemaphoreType.DMA,
        ],
    )
    return f(src1, dst1, src2, dst2, y1, y2)


def _y_call(x1, x2, dparts):
    return pl.pallas_call(
        _y_body,
        grid=(NRB,),
        in_specs=[
            pl.BlockSpec((RB, D), lambda r: (r, 0)),
            pl.BlockSpec((RB, D), lambda r: (r, 0)),
            pl.BlockSpec((2, NC, RB, 16), lambda r: (0, 0, r, 0)),
        ],
        out_specs=[
            pl.BlockSpec((RB, D), lambda r: (r, 0)),
            pl.BlockSpec((RB, D), lambda r: (r, 0)),
        ],
        out_shape=[
            jax.ShapeDtypeStruct((N, D), jnp.float32),
            jax.ShapeDtypeStruct((N, D), jnp.float32),
        ],
    )(x1, x2, dparts)


def _main_call(parts, y1, y2, dparts, bf, W_st, b_st, fcW_st, fcb_st,
               f1W, f1b, f2W, f2b, oW, ob):
    const2 = lambda r: (0, 0)
    const3 = lambda r: (0, 0, 0)
    return pl.pallas_call(
        _main_body,
        grid=(NRB,),
        in_specs=[
            pl.BlockSpec((2, NC, RB, D), lambda r: (0, 0, r, 0)),
            pl.BlockSpec((RB, D), lambda r: (r, 0)),
            pl.BlockSpec((RB, D), lambda r: (r, 0)),
            pl.BlockSpec((2, NC, RB, 16), lambda r: (0, 0, r, 0)),
            pl.BlockSpec((2, RB, 1), lambda r: (0, r, 0)),
            pl.BlockSpec((2, D, H), const3),
            pl.BlockSpec((2, 1, H), const3),
            pl.BlockSpec((2, H, OUT_DIM), const3),
            pl.BlockSpec((2, 1, OUT_DIM), const3),
            pl.BlockSpec((2 * OUT_DIM, OUT_DIM), const2),
            pl.BlockSpec((1, OUT_DIM), const2),
            pl.BlockSpec((OUT_DIM, 16), const2),
            pl.BlockSpec((1, 16), const2),
            pl.BlockSpec((16, 1), const2),
            pl.BlockSpec((1, 1), const2),
        ],
        out_specs=pl.BlockSpec((B, 1), const2),
        out_shape=jax.ShapeDtypeStruct((B, 1), jnp.float32),
        scratch_shapes=[
            pltpu.VMEM((2, B, H), jnp.float32),
            pltpu.VMEM((2, B, 8), jnp.float32),
        ],
    )(parts, y1, y2, dparts, bf, W_st, b_st, fcW_st, fcb_st,
      f1W, f1b, f2W, f2b, oW, ob)


def kernel(ch1_x, ch1_edge_index, ch1_batch, ch2_x, ch2_edge_index, ch2_batch,
           conv1_W, conv1_b, ch1_fc1_W, ch1_fc1_b,
           conv2_W, conv2_b, ch2_fc1_W, ch2_fc1_b,
           fc1_W, fc1_b, fc2_W, fc2_b, out_W, out_b):
    src1 = ch1_edge_index[0].reshape(E // CHUNK, CHUNK)
    dst1 = ch1_edge_index[1].reshape(E // CHUNK, CHUNK)
    src2 = ch2_edge_index[0].reshape(E // CHUNK, CHUNK)
    dst2 = ch2_edge_index[1].reshape(E // CHUNK, CHUNK)
    bf = jnp.stack([ch1_batch, ch2_batch]).astype(jnp.float32).reshape(2, N, 1)
    W_st = jnp.stack([conv1_W, conv2_W])
    b_st = jnp.stack([conv1_b, conv2_b]).reshape(2, 1, H)
    fcW_st = jnp.stack([ch1_fc1_W, ch2_fc1_W])
    fcb_st = jnp.stack([ch1_fc1_b, ch2_fc1_b]).reshape(2, 1, OUT_DIM)

    dparts = _deg_call(dst1, dst2)
    y1, y2 = _y_call(ch1_x, ch2_x, dparts)
    parts = _scatter_call(src1, dst1, src2, dst2, y1, y2)
    return _main_call(parts, y1, y2, dparts, bf, W_st, b_st, fcW_st, fcb_st,
                      fc1_W, fc1_b.reshape(1, OUT_DIM),
                      fc2_W, fc2_b.reshape(1, 16),
                      out_W, out_b.reshape(1, 1))


# speedup vs baseline: 1.2002x; 1.2002x over previous
import functools

import jax
import jax.numpy as jnp
from jax import lax
from jax.experimental import pallas as pl
from jax.experimental.pallas import tpu as pltpu
from jax.experimental.pallas import tpu_sc as plsc

N = 10000
E = 320000
D = 128
H = 640
B = 128
OUT_DIM = 64

NC = 2
NS = 16
NW = NC * NS
EPW = E // NW
CHUNK = 125
NCHUNK = EPW // CHUNK
NPAD = 10240
RPT = NPAD // NS
RB = 1000
NRB = N // RB
SLOPE = 0.01


def _lrelu(x):
    return jnp.where(x >= 0, x, SLOPE * x)


def _deg_body(dst1_h, dst2_h, deg_h, idx_v, ones_v, zero_v, deg_sh, dsem):
    cid = lax.axis_index("c")
    sid = lax.axis_index("s")
    wid = cid * NS + sid
    onerow = jnp.where(lax.iota(jnp.int32, 16) == 0,
                       jnp.full((16,), 1.0, jnp.float32),
                       jnp.zeros((16,), jnp.float32))
    zrow = jnp.zeros((16,), jnp.float32)

    def init_ones(i, carry):
        ones_v[i, :] = onerow
        return carry

    def init_zero(i, carry):
        zero_v[i, :] = zrow
        return carry

    lax.fori_loop(0, CHUNK, init_ones, 0)
    lax.fori_loop(0, 64, init_zero, 0)

    def zero_acc():
        for k in range(RPT // 64):
            pltpu.sync_copy(zero_v, deg_sh.at[pl.ds(sid * RPT + k * 64, 64)])

    zero_acc()
    plsc.subcore_barrier()

    for ch in range(2):
        dst_h = dst1_h if ch == 0 else dst2_h
        pltpu.sync_copy(dst_h.at[pl.ds(wid * NCHUNK, NCHUNK)], idx_v)

        def chunk_fire(j, carry):
            pltpu.async_copy(ones_v, deg_sh.at[idx_v.at[j]], dsem, add=True)
            return carry

        def chunk_drain(j, carry):
            pltpu.make_async_copy(ones_v, deg_sh.at[idx_v.at[j]],
                                  dsem).wait()
            return carry

        lax.fori_loop(0, NCHUNK, chunk_fire, 0)
        lax.fori_loop(0, NCHUNK, chunk_drain, 0)
        plsc.subcore_barrier()
        pltpu.sync_copy(deg_sh.at[pl.ds(sid * RPT, RPT)],
                        deg_h.at[ch, cid, pl.ds(sid * RPT, RPT)])
        plsc.subcore_barrier()
        if ch == 0:
            zero_acc()
            plsc.subcore_barrier()


def _scatter_body(src1_h, dst1_h, src2_h, dst2_h, y1_h, y2_h, part_h,
                  sidx_v, didx_v, rows0_v, rows1_v, zero_v, acc_sh,
                  sem0, sem1):
    cid = lax.axis_index("c")
    sid = lax.axis_index("s")
    wid = cid * NS + sid
    zrow = jnp.zeros((16,), jnp.float32)

    def zero_rows(i, carry):
        for jc in range(D // 16):
            zero_v[i, pl.ds(jc * 16, 16)] = zrow
        return carry

    lax.fori_loop(0, 16, zero_rows, 0)

    def zero_acc():
        for k in range(RPT // 16):
            pltpu.sync_copy(zero_v, acc_sh.at[pl.ds(sid * RPT + k * 16, 16)])

    zero_acc()
    plsc.subcore_barrier()

    for ch in range(2):
        src_h = src1_h if ch == 0 else src2_h
        dst_h = dst1_h if ch == 0 else dst2_h
        y_h = y1_h if ch == 0 else y2_h
        for half in range(2):
            hbase = wid * NCHUNK + half * (NCHUNK // 2)
            pltpu.sync_copy(src_h.at[pl.ds(hbase, NCHUNK // 2)], sidx_v)
            pltpu.sync_copy(dst_h.at[pl.ds(hbase, NCHUNK // 2)], didx_v)

            npair = NCHUNK // 4
            pltpu.async_copy(y_h.at[sidx_v.at[0]], rows0_v, sem0)
            pltpu.async_copy(y_h.at[sidx_v.at[1]], rows1_v, sem1)

            def pair_body(k, carry):
                j0 = 2 * k
                pltpu.make_async_copy(
                    y_h.at[sidx_v.at[j0]], rows0_v, sem0).wait()
                pltpu.sync_copy(rows0_v, acc_sh.at[didx_v.at[j0]], add=True)
                pltpu.async_copy(y_h.at[sidx_v.at[j0 + 2]], rows0_v, sem0)
                pltpu.make_async_copy(
                    y_h.at[sidx_v.at[j0 + 1]], rows1_v, sem1).wait()
                pltpu.sync_copy(
                    rows1_v, acc_sh.at[didx_v.at[j0 + 1]], add=True)
                pltpu.async_copy(y_h.at[sidx_v.at[j0 + 3]], rows1_v, sem1)
                return carry

            lax.fori_loop(0, npair - 1, pair_body, 0)
            jlast = NCHUNK // 2 - 2
            pltpu.make_async_copy(
                y_h.at[sidx_v.at[jlast]], rows0_v, sem0).wait()
            pltpu.sync_copy(rows0_v, acc_sh.at[didx_v.at[jlast]], add=True)
            pltpu.make_async_copy(
                y_h.at[sidx_v.at[jlast + 1]], rows1_v, sem1).wait()
            pltpu.sync_copy(
                rows1_v, acc_sh.at[didx_v.at[jlast + 1]], add=True)
        plsc.subcore_barrier()
        pltpu.sync_copy(acc_sh.at[pl.ds(sid * RPT, RPT)],
                        part_h.at[ch, cid, pl.ds(sid * RPT, RPT)])
        plsc.subcore_barrier()
        if ch == 0:
            zero_acc()
            plsc.subcore_barrier()


def _y_body(x1_ref, x2_ref, dp_ref, y1_ref, y2_ref):
    dp = dp_ref[...]
    for ch in range(2):
        deg = dp[ch, 0, :, 0:1] + dp[ch, 1, :, 0:1] + 1.0
        dinv = lax.rsqrt(deg)
        x = (x1_ref if ch == 0 else x2_ref)[...]
        (y1_ref if ch == 0 else y2_ref)[...] = x * dinv


def _main_body(part_ref, y1_ref, y2_ref, dp_ref, bf_ref, W_ref, b_ref,
               fcW_ref, fcb_ref, f1W_ref, f1b_ref, f2W_ref, f2b_ref,
               oW_ref, ob_ref, out_ref, pooled_sc, cnt_sc):
    r = pl.program_id(0)

    @pl.when(r == 0)
    def _init():
        pooled_sc[...] = jnp.zeros_like(pooled_sc)
        cnt_sc[...] = jnp.zeros_like(cnt_sc)

    iota_g = lax.broadcasted_iota(jnp.int32, (RB, B), 1).astype(jnp.float32)
    ones_col = jnp.ones((RB, 8), jnp.float32)
    dp = dp_ref[...]
    for ch in range(2):
        yv = (y1_ref if ch == 0 else y2_ref)[...]
        deg = dp[ch, 0, :, 0:1] + dp[ch, 1, :, 0:1] + 1.0
        dinv = lax.rsqrt(deg)
        agg = dinv * (part_ref[ch, 0] + part_ref[ch, 1] + yv)
        h = jnp.dot(agg, W_ref[ch], preferred_element_type=jnp.float32)
        z = _lrelu(h + b_ref[ch])
        oh = jnp.where(bf_ref[ch] == iota_g, 1.0, 0.0)
        pooled_sc[ch] += lax.dot_general(
            oh, z, (((0,), (0,)), ((), ())), preferred_element_type=jnp.float32)
        cnt_sc[ch] += lax.dot_general(
            oh, ones_col, (((0,), (0,)), ((), ())),
            preferred_element_type=jnp.float32)

    @pl.when(r == NRB - 1)
    def _final():
        feats = []
        for ch in range(2):
            cnt = jnp.maximum(cnt_sc[ch, :, 0:1], 1.0)
            mean = pooled_sc[ch] / cnt
            t = jnp.dot(mean, fcW_ref[ch],
                        preferred_element_type=jnp.float32) + fcb_ref[ch]
            feats.append(_lrelu(t))
        xc = jnp.concatenate(feats, axis=1)
        u = _lrelu(jnp.dot(xc, f1W_ref[...],
                           preferred_element_type=jnp.float32) + f1b_ref[...])
        v = _lrelu(jnp.dot(u, f2W_ref[...],
                           preferred_element_type=jnp.float32) + f2b_ref[...])
        out_ref[...] = jnp.dot(v, oW_ref[...],
                               preferred_element_type=jnp.float32) + ob_ref[...]


def _sc_mesh():
    return plsc.VectorSubcoreMesh(core_axis_name="c", subcore_axis_name="s",
                                  num_cores=NC, num_subcores=NS)


def _deg_call(dst1, dst2):
    f = pl.kernel(
        _deg_body,
        out_type=jax.ShapeDtypeStruct((2, NC, NPAD, 16), jnp.float32),
        mesh=_sc_mesh(),
        scratch_types=[
            pltpu.VMEM((NCHUNK, CHUNK), jnp.int32),
            pltpu.VMEM((CHUNK, 16), jnp.float32),
            pltpu.VMEM((64, 16), jnp.float32),
            pltpu.VMEM_SHARED((NPAD, 16), jnp.float32),
            pltpu.SemaphoreType.DMA,
        ],
    )
    return f(dst1, dst2)


def _scatter_call(src1, dst1, src2, dst2, y1, y2):
    f = pl.kernel(
        _scatter_body,
        out_type=jax.ShapeDtypeStruct((2, NC, NPAD, D), jnp.float32),
        mesh=_sc_mesh(),
        scratch_types=[
            pltpu.VMEM((NCHUNK // 2, CHUNK), jnp.int32),
            pltpu.VMEM((NCHUNK // 2, CHUNK), jnp.int32),
            pltpu.VMEM((CHUNK, D), jnp.float32),
            pltpu.VMEM((CHUNK, D), jnp.float32),
            pltpu.VMEM((16, D), jnp.float32),
            pltpu.VMEM_SHARED((NPAD, D), jnp.float32),
            pltpu.SemaphoreType.DMA,
            pltpu.SemaphoreType.DMA,
        ],
    )
    return f(src1, dst1, src2, dst2, y1, y2)


def _y_call(x1, x2, dparts):
    return pl.pallas_call(
        _y_body,
        grid=(NRB,),
        in_specs=[
            pl.BlockSpec((RB, D), lambda r: (r, 0)),
            pl.BlockSpec((RB, D), lambda r: (r, 0)),
            pl.BlockSpec((2, NC, RB, 16), lambda r: (0, 0, r, 0)),
        ],
        out_specs=[
            pl.BlockSpec((RB, D), lambda r: (r, 0)),
            pl.BlockSpec((RB, D), lambda r: (r, 0)),
        ],
        out_shape=[
            jax.ShapeDtypeStruct((N, D), jnp.float32),
            jax.ShapeDtypeStruct((N, D), jnp.float32),
        ],
    )(x1, x2, dparts)


def _main_call(parts, y1, y2, dparts, bf, W_st, b_st, fcW_st, fcb_st,
               f1W, f1b, f2W, f2b, oW, ob):
    const2 = lambda r: (0, 0)
    const3 = lambda r: (0, 0, 0)
    return pl.pallas_call(
        _main_body,
        grid=(NRB,),
        in_specs=[
            pl.BlockSpec((2, NC, RB, D), lambda r: (0, 0, r, 0)),
            pl.BlockSpec((RB, D), lambda r: (r, 0)),
            pl.BlockSpec((RB, D), lambda r: (r, 0)),
            pl.BlockSpec((2, NC, RB, 16), lambda r: (0, 0, r, 0)),
            pl.BlockSpec((2, RB, 1), lambda r: (0, r, 0)),
            pl.BlockSpec((2, D, H), const3),
            pl.BlockSpec((2, 1, H), const3),
            pl.BlockSpec((2, H, OUT_DIM), const3),
            pl.BlockSpec((2, 1, OUT_DIM), const3),
            pl.BlockSpec((2 * OUT_DIM, OUT_DIM), const2),
            pl.BlockSpec((1, OUT_DIM), const2),
            pl.BlockSpec((OUT_DIM, 16), const2),
            pl.BlockSpec((1, 16), const2),
            pl.BlockSpec((16, 1), const2),
            pl.BlockSpec((1, 1), const2),
        ],
        out_specs=pl.BlockSpec((B, 1), const2),
        out_shape=jax.ShapeDtypeStruct((B, 1), jnp.float32),
        scratch_shapes=[
            pltpu.VMEM((2, B, H), jnp.float32),
            pltpu.VMEM((2, B, 8), jnp.float32),
        ],
    )(parts, y1, y2, dparts, bf, W_st, b_st, fcW_st, fcb_st,
      f1W, f1b, f2W, f2b, oW, ob)


def kernel(ch1_x, ch1_edge_index, ch1_batch, ch2_x, ch2_edge_index, ch2_batch,
           conv1_W, conv1_b, ch1_fc1_W, ch1_fc1_b,
           conv2_W, conv2_b, ch2_fc1_W, ch2_fc1_b,
           fc1_W, fc1_b, fc2_W, fc2_b, out_W, out_b):
    src1 = ch1_edge_index[0].reshape(E // CHUNK, CHUNK)
    dst1 = ch1_edge_index[1].reshape(E // CHUNK, CHUNK)
    src2 = ch2_edge_index[0].reshape(E // CHUNK, CHUNK)
    dst2 = ch2_edge_index[1].reshape(E // CHUNK, CHUNK)
    bf = jnp.stack([ch1_batch, ch2_batch]).astype(jnp.float32).reshape(2, N, 1)
    W_st = jnp.stack([conv1_W, conv2_W])
    b_st = jnp.stack([conv1_b, conv2_b]).reshape(2, 1, H)
    fcW_st = jnp.stack([ch1_fc1_W, ch2_fc1_W])
    fcb_st = jnp.stack([ch1_fc1_b, ch2_fc1_b]).reshape(2, 1, OUT_DIM)

    dparts = _deg_call(dst1, dst2)
    y1, y2 = _y_call(ch1_x, ch2_x, dparts)
    parts = _scatter_call(src1, dst1, src2, dst2, y1, y2)
    return _main_call(parts, y1, y2, dparts, bf, W_st, b_st, fcW_st, fcb_st,
                      fc1_W, fc1_b.reshape(1, OUT_DIM),
                      fc2_W, fc2_b.reshape(1, 16),
                      out_W, out_b.reshape(1, 1))
